# baseline (speedup 1.0000x reference)
import functools

import jax
import jax.numpy as jnp
from jax import lax
from jax.experimental import pallas as pl
from jax.experimental.pallas import tpu as pltpu
from jax.experimental.pallas import tpu_sc as plsc

_NC = 2
_NS = 16
_NW = _NC * _NS
_L = 16
_D = 1024
_BATCH = 4
_NB_SC = 2
_SEQ = 2048
_ROWS_W = _SEQ // _NW
_PASS_ROWS = 32
_N_PASS = _ROWS_W // _PASS_ROWS
_CH = 16
_CPB = _PASS_ROWS // _CH
_T = _NB_SC * _CPB
_NIN = 3
_NOUT = 2
_SLICES = _CH * (_D // _L)


def _sc_body(x_hbm, pe_hbm, o_hbm, pe_v, x_bufs, o_bufs,
             is0, is1, is2, os0, os1):
    cid = lax.axis_index("c")
    sid = lax.axis_index("s")
    wid = sid * _NC + cid
    band0 = wid * _ROWS_W
    in_sems = (is0, is1, is2)
    out_sems = (os0, os1)

    def pass_body(p, carry):
        pe_row0 = band0 + p * _PASS_ROWS

        def xrow(t):
            b = t // _CPB
            c = t % _CPB
            return b * _SEQ + pe_row0 + c * _CH

        def start_in(t):
            s = t % _NIN
            return pltpu.async_copy(
                x_hbm.at[pl.ds(xrow(t), _CH)], x_bufs.at[s], in_sems[s])

        in_d = {}
        out_d = {}
        in_d[0] = start_in(0)
        in_d[1] = start_in(1)
        pltpu.sync_copy(pe_hbm.at[pl.ds(pe_row0, _PASS_ROWS)], pe_v)

        for t in range(_T):
            si = t % _NIN
            so = t % _NOUT
            if t + 2 < _T:
                in_d[t + 2] = start_in(t + 2)
            in_d[t].wait()
            if t >= _NOUT:
                out_d[t - _NOUT].wait()
            c = t % _CPB

            @plsc.parallel_loop(0, _SLICES, unroll=8)
            def _(i):
                r = i >> 6
                sl = pl.ds((i & 63) * _L, _L)
                o_bufs[so, r, sl] = x_bufs[si, r, sl] + pe_v[c * _CH + r, sl]

            out_d[t] = pltpu.async_copy(
                o_bufs.at[so], o_hbm.at[pl.ds(xrow(t), _CH)], out_sems[so])
        for t in range(_T - _NOUT, _T):
            out_d[t].wait()
        return carry

    lax.fori_loop(0, _N_PASS, pass_body, 0)


_sc_kernel = functools.partial(
    pl.kernel,
    out_type=jax.ShapeDtypeStruct((_NB_SC * _SEQ, _D), jnp.float32),
    mesh=plsc.VectorSubcoreMesh(core_axis_name="c", subcore_axis_name="s"),
    scratch_types=[
        pltpu.VMEM((_PASS_ROWS, _D), jnp.float32),
        pltpu.VMEM((_NIN, _CH, _D), jnp.float32),
        pltpu.VMEM((_NOUT, _CH, _D), jnp.float32),
        pltpu.SemaphoreType.DMA,
        pltpu.SemaphoreType.DMA,
        pltpu.SemaphoreType.DMA,
        pltpu.SemaphoreType.DMA,
        pltpu.SemaphoreType.DMA,
    ],
)(_sc_body)


def _tc_body(x_ref, pe_ref, o_ref):
    o_ref[...] = x_ref[...] + pe_ref[...][None, :, :]


def _tc_kernel(x, pe_weight):
    B, S, D = x.shape
    nb = B - _NB_SC
    bs = 256
    return pl.pallas_call(
        _tc_body,
        grid=(S // bs, nb),
        in_specs=[
            pl.BlockSpec((1, bs, D), lambda s, b: (b + _NB_SC, s, 0)),
            pl.BlockSpec((bs, D), lambda s, b: (s, 0)),
        ],
        out_specs=pl.BlockSpec((1, bs, D), lambda s, b: (b, s, 0)),
        out_shape=jax.ShapeDtypeStruct((nb, S, D), x.dtype),
    )(x, pe_weight)


def kernel(x, pe_weight):
    B, S, D = x.shape
    so = _sc_kernel(x.reshape(B * S, D), pe_weight)
    to = _tc_kernel(x, pe_weight)
    return jnp.concatenate([so.reshape(_NB_SC, S, D), to], axis=0)


# speedup vs baseline: 1.3654x; 1.3654x over previous
import functools

import jax
import jax.numpy as jnp
from jax import lax
from jax.experimental import pallas as pl
from jax.experimental.pallas import tpu as pltpu
from jax.experimental.pallas import tpu_sc as plsc

_NC = 2
_NS = 16
_NW = _NC * _NS
_L = 16
_D = 1024
_BATCH = 4
_SEQ = 2048
_ROWS_W = _SEQ // _NW
_PASS_ROWS = 64
_N_PASS = _ROWS_W // _PASS_ROWS
_CH = 16
_CPB = _PASS_ROWS // _CH
_T = _BATCH * _CPB
_NBUF = 3
_SLICES = _CH * (_D // _L)


def _sc_body(x_hbm, pe_hbm, o_hbm, pe_v, x_bufs,
             is0, is1, is2, is3, os0, os1, os2, os3):
    cid = lax.axis_index("c")
    sid = lax.axis_index("s")
    wid = sid * _NC + cid
    band0 = wid * _ROWS_W
    in_sems = (is0, is1, is2, is3)
    out_sems = (os0, os1, os2, os3)

    def pass_body(p, carry):
        pe_row0 = band0 + p * _PASS_ROWS

        def xrow(t):
            b = t // _CPB
            c = t % _CPB
            return b * _SEQ + pe_row0 + c * _CH

        def start_in(t):
            s = t % _NBUF
            return pltpu.async_copy(
                x_hbm.at[pl.ds(xrow(t), _CH)], x_bufs.at[s], in_sems[s])

        in_d = {}
        out_d = {}
        in_d[0] = start_in(0)
        in_d[1] = start_in(1)
        pltpu.sync_copy(pe_hbm.at[pl.ds(pe_row0, _PASS_ROWS)], pe_v)

        for t in range(_T):
            s = t % _NBUF
            in_d[t].wait()
            c = t % _CPB

            @plsc.parallel_loop(0, _SLICES, unroll=8)
            def _(i):
                r = i >> 6
                sl = pl.ds((i & 63) * _L, _L)
                v = pe_v[c * _CH + r, sl]
                plsc.addupdate(x_bufs.at[s, r, sl], v)

            out_d[t] = pltpu.async_copy(
                x_bufs.at[s], o_hbm.at[pl.ds(xrow(t), _CH)], out_sems[s])
            if t + 2 < _T:
                if t >= 1:
                    out_d[t - 1].wait()
                in_d[t + 2] = start_in(t + 2)
        for t in range(_T - _NBUF, _T):
            out_d[t].wait()
        return carry

    lax.fori_loop(0, _N_PASS, pass_body, 0)


_sc_kernel = functools.partial(
    pl.kernel,
    out_type=jax.ShapeDtypeStruct((_BATCH * _SEQ, _D), jnp.float32),
    mesh=plsc.VectorSubcoreMesh(core_axis_name="c", subcore_axis_name="s"),
    scratch_types=[
        pltpu.VMEM((_PASS_ROWS, _D), jnp.float32),
        pltpu.VMEM((_NBUF, _CH, _D), jnp.float32),
        pltpu.SemaphoreType.DMA,
        pltpu.SemaphoreType.DMA,
        pltpu.SemaphoreType.DMA,
        pltpu.SemaphoreType.DMA,
        pltpu.SemaphoreType.DMA,
        pltpu.SemaphoreType.DMA,
        pltpu.SemaphoreType.DMA,
        pltpu.SemaphoreType.DMA,
    ],
)(_sc_body)


def kernel(x, pe_weight):
    B, S, D = x.shape
    out = _sc_kernel(x.reshape(B * S, D), pe_weight)
    return out.reshape(B, S, D)


# speedup vs baseline: 1.4035x; 1.0279x over previous
import functools

import jax
import jax.numpy as jnp
from jax import lax
from jax.experimental import pallas as pl
from jax.experimental.pallas import tpu as pltpu
from jax.experimental.pallas import tpu_sc as plsc

_NC = 2
_NS = 16
_NW = _NC * _NS
_L = 16
_D = 1024
_BATCH = 4
_SEQ = 2048
_ROWS_W = _SEQ // _NW
_PASS_ROWS = 32
_N_PASS = _ROWS_W // _PASS_ROWS
_CH = 16
_CPB = _PASS_ROWS // _CH
_T = _BATCH * _CPB
_NBUF = 4
_SLICES = _CH * (_D // _L)


def _sc_body(x_hbm, pe_hbm, o_hbm, pe_v, x_bufs,
             is0, is1, is2, is3, os0, os1, os2, os3):
    cid = lax.axis_index("c")
    sid = lax.axis_index("s")
    wid = sid * _NC + cid
    band0 = wid * _ROWS_W
    in_sems = (is0, is1, is2, is3)
    out_sems = (os0, os1, os2, os3)

    def pass_body(p, carry):
        pe_row0 = band0 + p * _PASS_ROWS

        def xrow(t):
            b = t // _CPB
            c = t % _CPB
            return b * _SEQ + pe_row0 + c * _CH

        def start_in(t):
            s = t % _NBUF
            return pltpu.async_copy(
                x_hbm.at[pl.ds(xrow(t), _CH)], x_bufs.at[s], in_sems[s])

        in_d = {}
        out_d = {}
        in_d[0] = start_in(0)
        in_d[1] = start_in(1)
        pltpu.sync_copy(pe_hbm.at[pl.ds(pe_row0, _PASS_ROWS)], pe_v)

        for t in range(_T):
            s = t % _NBUF
            if t + 2 < _T:
                if t >= 2:
                    out_d[t - 2].wait()
                in_d[t + 2] = start_in(t + 2)
            in_d[t].wait()
            c = t % _CPB

            @plsc.parallel_loop(0, _SLICES // 2, unroll=4)
            def _(j):
                i = j * 2
                r = i >> 6
                k = (i & 63) * _L
                v0 = pe_v[c * _CH + r, pl.ds(k, _L)]
                plsc.addupdate(x_bufs.at[s, r, pl.ds(k, _L)], v0)
                v1 = pe_v[c * _CH + r, pl.ds(k + _L, _L)]
                plsc.addupdate(x_bufs.at[s, r, pl.ds(k + _L, _L)], v1)

            out_d[t] = pltpu.async_copy(
                x_bufs.at[s], o_hbm.at[pl.ds(xrow(t), _CH)], out_sems[s])
        for t in range(_T - _NBUF, _T):
            out_d[t].wait()
        return carry

    lax.fori_loop(0, _N_PASS, pass_body, 0)


_sc_kernel = functools.partial(
    pl.kernel,
    out_type=jax.ShapeDtypeStruct((_BATCH * _SEQ, _D), jnp.float32),
    mesh=plsc.VectorSubcoreMesh(core_axis_name="c", subcore_axis_name="s"),
    scratch_types=[
        pltpu.VMEM((_PASS_ROWS, _D), jnp.float32),
        pltpu.VMEM((_NBUF, _CH, _D), jnp.float32),
        pltpu.SemaphoreType.DMA,
        pltpu.SemaphoreType.DMA,
        pltpu.SemaphoreType.DMA,
        pltpu.SemaphoreType.DMA,
        pltpu.SemaphoreType.DMA,
        pltpu.SemaphoreType.DMA,
        pltpu.SemaphoreType.DMA,
        pltpu.SemaphoreType.DMA,
    ],
)(_sc_body)


def kernel(x, pe_weight):
    B, S, D = x.shape
    out = _sc_kernel(x.reshape(B * S, D), pe_weight)
    return out.reshape(B, S, D)


# speedup vs baseline: 1.4100x; 1.0047x over previous
import functools

import jax
import jax.numpy as jnp
from jax import lax
from jax.experimental import pallas as pl
from jax.experimental.pallas import tpu as pltpu
from jax.experimental.pallas import tpu_sc as plsc

_NC = 2
_NS = 16
_NW = _NC * _NS
_L = 16
_D = 1024
_BATCH = 4
_SEQ = 2048
_ROWS_W = _SEQ // _NW
_PASS_ROWS = 32
_N_PASS = _ROWS_W // _PASS_ROWS
_CH = 16
_CPB = _PASS_ROWS // _CH
_T = _BATCH * _CPB
_NBUF = 4
_SLICES = _CH * (_D // _L)


def _sc_body(x_hbm, pe_hbm, o_hbm, pe_v, x_bufs,
             is0, is1, is2, is3, os0, os1, os2, os3):
    cid = lax.axis_index("c")
    sid = lax.axis_index("s")
    wid = sid * _NC + cid
    band0 = wid * _ROWS_W
    in_sems = (is0, is1, is2, is3)
    out_sems = (os0, os1, os2, os3)

    def pass_body(p, carry):
        pe_row0 = band0 + p * _PASS_ROWS

        def xrow(t):
            b = t // _CPB
            c = t % _CPB
            return b * _SEQ + pe_row0 + c * _CH

        def start_in(t):
            s = t % _NBUF
            return pltpu.async_copy(
                x_hbm.at[pl.ds(xrow(t), _CH)], x_bufs.at[s], in_sems[s])

        in_d = {}
        out_d = {}
        in_d[0] = start_in(0)
        in_d[1] = start_in(1)
        pltpu.sync_copy(pe_hbm.at[pl.ds(pe_row0, _PASS_ROWS)], pe_v)

        for t in range(_T):
            s = t % _NBUF
            if t + 2 < _T:
                if t >= 2:
                    out_d[t - 2].wait()
                in_d[t + 2] = start_in(t + 2)
            in_d[t].wait()
            c = t % _CPB

            @plsc.parallel_loop(0, _SLICES, unroll=8)
            def _(i):
                r = i >> 6
                sl = pl.ds((i & 63) * _L, _L)
                v = pe_v[c * _CH + r, sl]
                plsc.addupdate(x_bufs.at[s, r, sl], v)

            out_d[t] = pltpu.async_copy(
                x_bufs.at[s], o_hbm.at[pl.ds(xrow(t), _CH)], out_sems[s])
        for t in range(_T - _NBUF, _T):
            out_d[t].wait()
        return carry

    lax.fori_loop(0, _N_PASS, pass_body, 0)


_sc_kernel = functools.partial(
    pl.kernel,
    out_type=jax.ShapeDtypeStruct((_BATCH * _SEQ, _D), jnp.float32),
    mesh=plsc.VectorSubcoreMesh(core_axis_name="c", subcore_axis_name="s"),
    scratch_types=[
        pltpu.VMEM((_PASS_ROWS, _D), jnp.float32),
        pltpu.VMEM((_NBUF, _CH, _D), jnp.float32),
        pltpu.SemaphoreType.DMA,
        pltpu.SemaphoreType.DMA,
        pltpu.SemaphoreType.DMA,
        pltpu.SemaphoreType.DMA,
        pltpu.SemaphoreType.DMA,
        pltpu.SemaphoreType.DMA,
        pltpu.SemaphoreType.DMA,
        pltpu.SemaphoreType.DMA,
    ],
)(_sc_body)


def kernel(x, pe_weight):
    B, S, D = x.shape
    out = _sc_kernel(x.reshape(B * S, D), pe_weight)
    return out.reshape(B, S, D)


# speedup vs baseline: 1.4118x; 1.0013x over previous
import functools

import jax
import jax.numpy as jnp
from jax import lax
from jax.experimental import pallas as pl
from jax.experimental.pallas import tpu as pltpu
from jax.experimental.pallas import tpu_sc as plsc

_NC = 2
_NS = 16
_NW = _NC * _NS
_L = 16
_D = 1024
_BATCH = 4
_SEQ = 2048
_ROWS_W = _SEQ // _NW
_PASS_ROWS = 32
_N_PASS = _ROWS_W // _PASS_ROWS
_CH = 16
_CPB = _PASS_ROWS // _CH
_T = _BATCH * _CPB
_NBUF = 4
_SLICES = _CH * (_D // _L)


def _sc_body(x_hbm, pe_hbm, o_hbm, pe_v, x_bufs,
             is0, is1, is2, is3, os0, os1, os2, os3):
    cid = lax.axis_index("c")
    sid = lax.axis_index("s")
    wid = sid * _NC + cid
    band0 = wid * _ROWS_W
    in_sems = (is0, is1, is2, is3)
    out_sems = (os0, os1, os2, os3)

    def pass_body(p, carry):
        pe_row0 = band0 + p * _PASS_ROWS

        def xrow(t):
            b = t // _CPB
            c = t % _CPB
            return b * _SEQ + pe_row0 + c * _CH

        def start_in(t):
            s = t % _NBUF
            return pltpu.async_copy(
                x_hbm.at[pl.ds(xrow(t), _CH)], x_bufs.at[s], in_sems[s])

        in_d = {}
        out_d = {}
        in_d[0] = start_in(0)
        in_d[1] = start_in(1)
        pltpu.sync_copy(pe_hbm.at[pl.ds(pe_row0, _PASS_ROWS)], pe_v)

        in_d[2] = start_in(2)
        for t in range(_T):
            s = t % _NBUF
            in_d[t].wait()
            c = t % _CPB

            @plsc.parallel_loop(0, _SLICES, unroll=8)
            def _(i):
                r = i >> 6
                sl = pl.ds((i & 63) * _L, _L)
                v = pe_v[c * _CH + r, sl]
                plsc.addupdate(x_bufs.at[s, r, sl], v)

            out_d[t] = pltpu.async_copy(
                x_bufs.at[s], o_hbm.at[pl.ds(xrow(t), _CH)], out_sems[s])
            if t + 3 < _T:
                if t >= 1:
                    out_d[t - 1].wait()
                in_d[t + 3] = start_in(t + 3)
        for t in range(_T - _NBUF, _T):
            out_d[t].wait()
        return carry

    lax.fori_loop(0, _N_PASS, pass_body, 0)


_sc_kernel = functools.partial(
    pl.kernel,
    out_type=jax.ShapeDtypeStruct((_BATCH * _SEQ, _D), jnp.float32),
    mesh=plsc.VectorSubcoreMesh(core_axis_name="c", subcore_axis_name="s"),
    scratch_types=[
        pltpu.VMEM((_PASS_ROWS, _D), jnp.float32),
        pltpu.VMEM((_NBUF, _CH, _D), jnp.float32),
        pltpu.SemaphoreType.DMA,
        pltpu.SemaphoreType.DMA,
        pltpu.SemaphoreType.DMA,
        pltpu.SemaphoreType.DMA,
        pltpu.SemaphoreType.DMA,
        pltpu.SemaphoreType.DMA,
        pltpu.SemaphoreType.DMA,
        pltpu.SemaphoreType.DMA,
    ],
)(_sc_body)


def kernel(x, pe_weight):
    B, S, D = x.shape
    out = _sc_kernel(x.reshape(B * S, D), pe_weight)
    return out.reshape(B, S, D)
